# baseline (speedup 1.0000x reference)
import functools

import jax
import jax.numpy as jnp
from jax import lax
from jax.experimental import pallas as pl
from jax.experimental.pallas import tpu as pltpu
from jax.experimental.pallas import tpu_sc as plsc

EMB = 64
SCALE = 8.0

NC = 2
NS = 16
NW = NC * NS
CHUNK = 128
VPC = CHUNK * EMB // 16


def _make_emb(n_total):
    n_per_w = n_total // NW
    n_chunks = n_per_w // CHUNK
    mesh = plsc.VectorSubcoreMesh(core_axis_name="c", subcore_axis_name="s")

    @functools.partial(
        pl.kernel,
        mesh=mesh,
        out_type=jax.ShapeDtypeStruct((n_total, EMB), jnp.float32),
        compiler_params=pltpu.CompilerParams(use_tc_tiling_on_sc=False),
        scratch_types=[
            pltpu.VMEM((n_chunks, CHUNK), jnp.int32),
            pltpu.VMEM((CHUNK, EMB), jnp.float32),
            pltpu.SemaphoreType.DMA,
        ],
    )
    def emb_kernel(idx_hbm, table_hbm, out_hbm, idx_v, rows, sem):
        wid = lax.axis_index("s") * NC + lax.axis_index("c")
        pltpu.sync_copy(idx_hbm.at[wid], idx_v)
        base = wid * n_per_w

        def chunk_body(j, _):
            pltpu.async_copy(table_hbm.at[idx_v.at[j]], rows, sem).wait()

            def scale_row(r, _):
                for c in range(EMB // 16):
                    sl = pl.ds(c * 16, 16)
                    rows[r, sl] = rows[r, sl] * SCALE
                return 0

            lax.fori_loop(0, CHUNK, scale_row, 0)
            pltpu.sync_copy(rows, out_hbm.at[pl.ds(base + j * CHUNK, CHUNK)])
            return 0

        lax.fori_loop(0, n_chunks, chunk_body, 0)

    return emb_kernel


def kernel(x, table):
    b, l = x.shape
    n_total = b * l
    idx = x.reshape(NW, n_total // (NW * CHUNK), CHUNK).astype(jnp.int32)
    out = _make_emb(n_total)(idx, table)
    return out.reshape(b, l, EMB)


# speedup vs baseline: 1.0807x; 1.0807x over previous
import functools

import jax
import jax.numpy as jnp
from jax import lax
from jax.experimental import pallas as pl
from jax.experimental.pallas import tpu as pltpu
from jax.experimental.pallas import tpu_sc as plsc

EMB = 64
SCALE = 8.0

NC = 2
NS = 16
NW = NC * NS
CHUNK = 128
NBUF = 5
VPC = CHUNK * EMB // 16


def _make_emb(n_total):
    n_per_w = n_total // NW
    n_chunks = n_per_w // CHUNK
    n_groups = n_chunks // NBUF
    mesh = plsc.VectorSubcoreMesh(core_axis_name="c", subcore_axis_name="s")

    @functools.partial(
        pl.kernel,
        mesh=mesh,
        out_type=jax.ShapeDtypeStruct((n_total, EMB), jnp.float32),
        compiler_params=pltpu.CompilerParams(use_tc_tiling_on_sc=False),
        scratch_types=[
            pltpu.VMEM((n_chunks, CHUNK), jnp.int32),
            [pltpu.VMEM((CHUNK, EMB), jnp.float32) for _ in range(NBUF)],
            [pltpu.SemaphoreType.DMA for _ in range(NBUF)],
            [pltpu.SemaphoreType.DMA for _ in range(NBUF)],
        ],
    )
    def emb_kernel(idx_hbm, table_hbm, out_hbm, idx_v, bufs, gsems, ssems):
        wid = lax.axis_index("s") * NC + lax.axis_index("c")
        pltpu.sync_copy(idx_hbm.at[wid], idx_v)
        base = wid * n_per_w

        def gather(j, b, sem):
            return pltpu.make_async_copy(table_hbm.at[idx_v.at[j]], bufs[b], sem)

        def store(j, b, sem):
            return pltpu.make_async_copy(
                bufs[b], out_hbm.at[pl.ds(base + j * CHUNK, CHUNK)], sem)

        for b in range(NBUF - 1):
            gather(b, b, gsems[b]).start()

        @pl.loop(0, n_groups)
        def group(g):
            for b in range(NBUF):
                j = g * NBUF + b
                gather(j, b, gsems[b]).wait()

                @plsc.parallel_loop(0, VPC, unroll=8)
                def scale(i):
                    r = i >> 2
                    sl = pl.ds((i & 3) * 16, 16)
                    bufs[b][r, sl] = bufs[b][r, sl] * SCALE

                store(j, b, ssems[b]).start()

                bn = (b + NBUF - 1) % NBUF

                @pl.when(j + NBUF - 1 < n_chunks)
                def _():
                    @pl.when(j >= 1)
                    def _():
                        store(j - 1, bn, ssems[bn]).wait()
                    gather(j + NBUF - 1, bn, gsems[bn]).start()

        for jj in range(n_chunks - NBUF, n_chunks):
            store(jj, jj % NBUF, ssems[jj % NBUF]).wait()

    return emb_kernel


def kernel(x, table):
    b, l = x.shape
    n_total = b * l
    idx = x.reshape(NW, n_total // (NW * CHUNK), CHUNK).astype(jnp.int32)
    out = _make_emb(n_total)(idx, table)
    return out.reshape(b, l, EMB)


# speedup vs baseline: 1.0822x; 1.0015x over previous
import functools

import jax
import jax.numpy as jnp
from jax import lax
from jax.experimental import pallas as pl
from jax.experimental.pallas import tpu as pltpu
from jax.experimental.pallas import tpu_sc as plsc

EMB = 64
SCALE = 8.0

NC = 2
NS = 16
NW = NC * NS
NBUF = 4
HALF = 2


def _make_emb(n_b, n_l):
    b_per_w = n_b // NW
    seg = n_l // HALF
    vpb = n_l * EMB // 16
    mesh = plsc.VectorSubcoreMesh(core_axis_name="c", subcore_axis_name="s")

    @functools.partial(
        pl.kernel,
        mesh=mesh,
        out_type=jax.ShapeDtypeStruct((n_b, n_l, EMB), jnp.float32),
        compiler_params=pltpu.CompilerParams(use_tc_tiling_on_sc=False),
        scratch_types=[
            pltpu.VMEM((b_per_w * HALF, seg), jnp.int32),
            [pltpu.VMEM((n_l, EMB), jnp.float32) for _ in range(NBUF)],
            [pltpu.SemaphoreType.DMA for _ in range(NBUF)],
            [pltpu.SemaphoreType.DMA for _ in range(NBUF)],
        ],
    )
    def emb_kernel(idx_hbm, table_hbm, out_hbm, idx_v, bufs, gsems, ssems):
        wid = lax.axis_index("s") * NC + lax.axis_index("c")
        base = wid * b_per_w
        pltpu.sync_copy(idx_hbm.at[pl.ds(base * HALF, b_per_w * HALF)], idx_v)

        def gather(j, b, sem):
            for k in range(HALF):
                pltpu.async_copy(
                    table_hbm.at[idx_v.at[j * HALF + k]],
                    bufs[b].at[pl.ds(k * seg, seg)], sem)

        def gather_wait(j, b, sem):
            for k in range(HALF):
                pltpu.make_async_copy(
                    table_hbm.at[idx_v.at[j * HALF + k]],
                    bufs[b].at[pl.ds(k * seg, seg)], sem).wait()

        def store(j, b, sem):
            return pltpu.make_async_copy(bufs[b], out_hbm.at[base + j], sem)

        for b in range(NBUF - 1):
            gather(b, b, gsems[b])

        @pl.loop(0, b_per_w // NBUF)
        def group(g):
            for b in range(NBUF):
                j = g * NBUF + b
                gather_wait(j, b, gsems[b])

                @plsc.parallel_loop(0, vpb, unroll=8)
                def scale(i):
                    r = i >> 2
                    sl = pl.ds((i & 3) * 16, 16)
                    bufs[b][r, sl] = bufs[b][r, sl] * SCALE

                store(j, b, ssems[b]).start()

                bn = (b + NBUF - 1) % NBUF

                @pl.when(j + NBUF - 1 < b_per_w)
                def _():
                    @pl.when(j >= 1)
                    def _():
                        store(j - 1, bn, ssems[bn]).wait()
                    gather(j + NBUF - 1, bn, gsems[bn])

        for jj in range(b_per_w - NBUF, b_per_w):
            store(jj, jj % NBUF, ssems[jj % NBUF]).wait()

    return emb_kernel


def kernel(x, table):
    n_b, n_l = x.shape
    idx = x.reshape(n_b * HALF, n_l // HALF).astype(jnp.int32)
    return _make_emb(n_b, n_l)(idx, table)


# speedup vs baseline: 1.0830x; 1.0007x over previous
import functools

import jax
import jax.numpy as jnp
from jax import lax
from jax.experimental import pallas as pl
from jax.experimental.pallas import tpu as pltpu
from jax.experimental.pallas import tpu_sc as plsc

EMB = 64
SCALE = 8.0

NC = 2
NS = 16
NW = NC * NS
NBUF = 4
HALF = 2


def _make_emb(n_b, n_l):
    b_per_w = n_b // NW
    segs = []
    off = 0
    while off < n_l:
        s = min(128, n_l - off)
        s -= s % 8
        segs.append((off, s))
        off += s
    vpb = n_l * EMB // 16
    mesh = plsc.VectorSubcoreMesh(core_axis_name="c", subcore_axis_name="s")

    @functools.partial(
        pl.kernel,
        mesh=mesh,
        out_type=jax.ShapeDtypeStruct((n_b, n_l, EMB), jnp.float32),
        compiler_params=pltpu.CompilerParams(use_tc_tiling_on_sc=False),
        scratch_types=[
            pltpu.VMEM((b_per_w, n_l), jnp.int32),
            [pltpu.VMEM((n_l, EMB), jnp.float32) for _ in range(NBUF)],
            [pltpu.SemaphoreType.DMA for _ in range(NBUF)],
            [pltpu.SemaphoreType.DMA for _ in range(NBUF)],
        ],
    )
    def emb_kernel(idx_hbm, table_hbm, out_hbm, idx_v, bufs, gsems, ssems):
        wid = lax.axis_index("s") * NC + lax.axis_index("c")
        base = wid * b_per_w
        pltpu.sync_copy(idx_hbm.at[pl.ds(base, b_per_w)], idx_v)

        def gather(j, b, sem):
            for off, s in segs:
                pltpu.async_copy(
                    table_hbm.at[idx_v.at[j, pl.ds(off, s)]],
                    bufs[b].at[pl.ds(off, s)], sem)

        def gather_wait(j, b, sem):
            for off, s in segs:
                pltpu.make_async_copy(
                    table_hbm.at[idx_v.at[j, pl.ds(off, s)]],
                    bufs[b].at[pl.ds(off, s)], sem).wait()

        def store(j, b, sem):
            return pltpu.make_async_copy(bufs[b], out_hbm.at[base + j], sem)

        for b in range(NBUF - 1):
            gather(b, b, gsems[b])

        @pl.loop(0, b_per_w // NBUF)
        def group(g):
            for b in range(NBUF):
                j = g * NBUF + b
                gather_wait(j, b, gsems[b])

                @plsc.parallel_loop(0, vpb, unroll=8)
                def scale(i):
                    r = i >> 2
                    sl = pl.ds((i & 3) * 16, 16)
                    bufs[b][r, sl] = bufs[b][r, sl] * SCALE

                store(j, b, ssems[b]).start()

                bn = (b + NBUF - 1) % NBUF

                @pl.when(j + NBUF - 1 < b_per_w)
                def _():
                    @pl.when(j >= 1)
                    def _():
                        store(j - 1, bn, ssems[bn]).wait()
                    gather(j + NBUF - 1, bn, gsems[bn])

        for jj in range(b_per_w - NBUF, b_per_w):
            store(jj, jj % NBUF, ssems[jj % NBUF]).wait()

    return emb_kernel


def kernel(x, table):
    n_b, n_l = x.shape
    return _make_emb(n_b, n_l)(x.astype(jnp.int32), table)
